# baseline (speedup 1.0000x reference)
import functools

import jax
import jax.numpy as jnp
from jax import lax
from jax.experimental import pallas as pl
from jax.experimental.pallas import tpu as pltpu
from jax.experimental.pallas import tpu_sc as plsc

_VOCAB = 1024
_HID = 64
_N = 1048576

_EDGE_PAD = 2048

_info = plsc.get_sparse_core_info()
_NC, _NS, _L = _info.num_cores, _info.num_subcores, _info.num_lanes
_NW = _NC * _NS
_PW = _N // _NW
_CH = 128
_HCH = 128
_SB = 8192
_NSB = _PW // _SB
_CPS = _SB // _CH


def _normalize_body(t_ref, o_ref):
    t = t_ref[...]
    ss = jnp.sum(t * t, axis=1, keepdims=True)
    norm = jnp.sqrt(ss)
    scale = jnp.where(norm > 1.0, 1.0 / norm, jnp.ones_like(norm))
    tn = (t * scale).astype(jnp.bfloat16)
    t3 = tn.reshape(_VOCAB, _HID // 2, 2)
    u = jax.lax.bitcast_convert_type(t3, jnp.uint16).astype(jnp.uint32)
    word = u[:, :, 0] | (u[:, :, 1] << 16)
    o_ref[...] = word.T.astype(jnp.int32)


def _normalize_table(table):
    return pl.pallas_call(
        _normalize_body,
        out_shape=jax.ShapeDtypeStruct((_HID // 2, _VOCAB), jnp.int32),
    )(table)


def _sc_body(vals_hbm, table_hbm, edges_hbm, emb_out, tok_out,
             edges_v, table_v, vals_v, toks_v, obuf0, obuf1, obuf2, obuf3,
             wsem0, wsem1, wsem2, wsem3):
    obuf = (obuf0, obuf1, obuf2, obuf3)
    wsem = (wsem0, wsem1, wsem2, wsem3)

    wid = lax.axis_index("s") * _NC + lax.axis_index("c")
    base = wid * _PW

    pltpu.sync_copy(edges_hbm, edges_v)
    pltpu.sync_copy(table_hbm, table_v)

    def fill_half(cl, b, half):
        toks = []
        for j in range(_HCH // _L):
            v = vals_v[pl.ds(cl * _CH + half * _HCH + j * _L, _L)]
            q = jnp.full((_L,), _VOCAB - 1, jnp.int32)
            k = _VOCAB
            while k >= 2:
                e = plsc.load_gather(edges_v, [q])
                m = e < v
                q = q + jnp.where(m, k // 2, -(k // 2))
                k //= 2
            e = plsc.load_gather(edges_v, [q])
            pos = q + jnp.where(e < v, 1, 0)
            toks.append(jnp.clip(pos - 1, 0, _VOCAB - 1))
        for j in range(_HCH // _L):
            toks_v[pl.ds(cl * _CH + half * _HCH + j * _L, _L)] = toks[j]
            col = half * _HCH + j * _L
            for g in range(0, _HID // 2, 8):
                words = [
                    plsc.load_gather(
                        table_v.at[pl.ds((g + t) * _VOCAB, _VOCAB)], [toks[j]])
                    for t in range(8)]
                for t in range(8):
                    lo, hi = plsc.unpack(
                        plsc.bitcast(words[t], jnp.bfloat16),
                        format=plsc.PackFormat.INTERLEAVED,
                        preferred_element_type=jnp.float32)
                    obuf[b][2 * (g + t), pl.ds(col, _L)] = lo
                    obuf[b][2 * (g + t) + 1, pl.ds(col, _L)] = hi

    def fill_chunk(cl, b):
        for half in range(_CH // _HCH):
            fill_half(cl, b, half)

    def wb_desc(sb, cl, b):
        gbase = base + sb * _SB + cl * _CH
        return pltpu.make_async_copy(
            obuf[b], emb_out.at[:, pl.ds(gbase, _CH)], wsem[b])

    def superblock(sb, carry):
        pltpu.sync_copy(vals_hbm.at[pl.ds(base + sb * _SB, _SB)], vals_v)

        def body(cp, carry2):
            for b in range(4):
                cl = cp * 4 + b

                @pl.when(cp > 0)
                def _():
                    wb_desc(sb, cl - 4, b).wait()

                fill_chunk(cl, b)
                wb_desc(sb, cl, b).start()
            return carry2

        lax.fori_loop(0, _CPS // 4, body, 0)

        for b in range(4):
            wb_desc(sb, _CPS - 4 + b, b).wait()

        pltpu.sync_copy(toks_v, tok_out.at[pl.ds(base + sb * _SB, _SB)])
        return carry

    lax.fori_loop(0, _NSB, superblock, 0)


_sc_lookup = functools.partial(
    pl.kernel,
    mesh=plsc.VectorSubcoreMesh(core_axis_name="c", subcore_axis_name="s"),
    out_type=[
        jax.ShapeDtypeStruct((_HID, _N), jnp.float32),
        jax.ShapeDtypeStruct((_N,), jnp.int32),
    ],
    scratch_types=[
        pltpu.VMEM((_EDGE_PAD,), jnp.float32),
        pltpu.VMEM((_HID // 2 * _VOCAB,), jnp.int32),
        pltpu.VMEM((_SB,), jnp.float32),
        pltpu.VMEM((_SB,), jnp.int32),
        pltpu.VMEM((_HID, _CH), jnp.float32),
        pltpu.VMEM((_HID, _CH), jnp.float32),
        pltpu.VMEM((_HID, _CH), jnp.float32),
        pltpu.VMEM((_HID, _CH), jnp.float32),
        pltpu.SemaphoreType.DMA,
        pltpu.SemaphoreType.DMA,
        pltpu.SemaphoreType.DMA,
        pltpu.SemaphoreType.DMA,
    ],
    compiler_params=pltpu.CompilerParams(
        needs_layout_passes=False, use_tc_tiling_on_sc=True),
)(_sc_body)


def kernel(ts_values, table, bin_edges):
    table_n = _normalize_table(table)
    edges = jnp.full((_EDGE_PAD,), jnp.inf, dtype=jnp.float32)
    edges = edges.at[: _VOCAB + 1].set(bin_edges)
    emb_t, toks = _sc_lookup(ts_values, table_n.reshape(-1), edges)
    return (emb_t.T, toks)


# speedup vs baseline: 1.1148x; 1.1148x over previous
import functools

import jax
import jax.numpy as jnp
from jax import lax
from jax.experimental import pallas as pl
from jax.experimental.pallas import tpu as pltpu
from jax.experimental.pallas import tpu_sc as plsc

_VOCAB = 1024
_HID = 64
_N = 1048576

_EDGE_PAD = 2048

_info = plsc.get_sparse_core_info()
_NC, _NS, _L = _info.num_cores, _info.num_subcores, _info.num_lanes
_NW = _NC * _NS
_PW = _N // _NW
_CH = 128
_HCH = 128
_SB = 8192
_NSB = _PW // _SB
_CPS = _SB // _CH


def _normalize_body(t_ref, o_ref):
    t = t_ref[...]
    ss = jnp.sum(t * t, axis=1, keepdims=True)
    norm = jnp.sqrt(ss)
    scale = jnp.where(norm > 1.0, 1.0 / norm, jnp.ones_like(norm))
    tn = (t * scale).astype(jnp.bfloat16)
    t3 = tn.reshape(_VOCAB, _HID // 2, 2)
    u = jax.lax.bitcast_convert_type(t3, jnp.uint16).astype(jnp.uint32)
    word = u[:, :, 0] | (u[:, :, 1] << 16)
    o_ref[...] = word.T.astype(jnp.int32)


def _normalize_table(table):
    return pl.pallas_call(
        _normalize_body,
        out_shape=jax.ShapeDtypeStruct((_HID // 2, _VOCAB), jnp.int32),
    )(table)


def _sc_body(vals_hbm, table_hbm, edges_hbm, emb_out, tok_out,
             edges_v, table_v, vals_v, toks_v, obuf0, obuf1, wsem0, wsem1):
    obuf = (obuf0, obuf1)
    wsem = (wsem0, wsem1)

    wid = lax.axis_index("s") * _NC + lax.axis_index("c")
    base = wid * _PW

    pltpu.sync_copy(edges_hbm, edges_v)
    pltpu.sync_copy(table_hbm, table_v)

    def fill_half(cl, b, half):
        toks = []
        for j in range(_HCH // _L):
            v = vals_v[pl.ds(cl * _CH + half * _HCH + j * _L, _L)]
            q = jnp.full((_L,), _VOCAB - 1, jnp.int32)
            k = _VOCAB
            while k >= 2:
                e = plsc.load_gather(edges_v, [q])
                m = e < v
                q = q + jnp.where(m, k // 2, -(k // 2))
                k //= 2
            e = plsc.load_gather(edges_v, [q])
            pos = q + jnp.where(e < v, 1, 0)
            toks.append(jnp.clip(pos - 1, 0, _VOCAB - 1))
        for j in range(_HCH // _L):
            toks_v[pl.ds(cl * _CH + half * _HCH + j * _L, _L)] = toks[j]
            col = half * _HCH + j * _L
            for g in range(0, _HID // 2, 8):
                words = [
                    plsc.load_gather(
                        table_v.at[pl.ds((g + t) * _VOCAB, _VOCAB)], [toks[j]])
                    for t in range(8)]
                for t in range(8):
                    lo, hi = plsc.unpack(
                        plsc.bitcast(words[t], jnp.bfloat16),
                        format=plsc.PackFormat.INTERLEAVED,
                        preferred_element_type=jnp.float32)
                    obuf[b][2 * (g + t), pl.ds(col, _L)] = lo
                    obuf[b][2 * (g + t) + 1, pl.ds(col, _L)] = hi

    def fill_chunk(cl, b):
        for half in range(_CH // _HCH):
            fill_half(cl, b, half)

    def wb_desc(sb, cl, b):
        gbase = base + sb * _SB + cl * _CH
        return pltpu.make_async_copy(
            obuf[b], emb_out.at[:, pl.ds(gbase, _CH)], wsem[b])

    def superblock(sb, carry):
        pltpu.sync_copy(vals_hbm.at[pl.ds(base + sb * _SB, _SB)], vals_v)

        def body(cp, carry2):
            for b in range(2):
                cl = cp * 2 + b

                @pl.when(cp > 0)
                def _():
                    wb_desc(sb, cl - 2, b).wait()

                fill_chunk(cl, b)
                wb_desc(sb, cl, b).start()
            return carry2

        lax.fori_loop(0, _CPS // 2, body, 0)

        for b in range(2):
            wb_desc(sb, _CPS - 2 + b, b).wait()

        pltpu.sync_copy(toks_v, tok_out.at[pl.ds(base + sb * _SB, _SB)])
        return carry

    lax.fori_loop(0, _NSB, superblock, 0)


_sc_lookup = functools.partial(
    pl.kernel,
    mesh=plsc.VectorSubcoreMesh(core_axis_name="c", subcore_axis_name="s"),
    out_type=[
        jax.ShapeDtypeStruct((_HID, _N), jnp.float32),
        jax.ShapeDtypeStruct((_N,), jnp.int32),
    ],
    scratch_types=[
        pltpu.VMEM((_EDGE_PAD,), jnp.float32),
        pltpu.VMEM((_HID // 2 * _VOCAB,), jnp.int32),
        pltpu.VMEM((_SB,), jnp.float32),
        pltpu.VMEM((_SB,), jnp.int32),
        pltpu.VMEM((_HID, _CH), jnp.float32),
        pltpu.VMEM((_HID, _CH), jnp.float32),
        pltpu.SemaphoreType.DMA,
        pltpu.SemaphoreType.DMA,
    ],
    compiler_params=pltpu.CompilerParams(
        needs_layout_passes=False, use_tc_tiling_on_sc=True),
)(_sc_body)


def kernel(ts_values, table, bin_edges):
    table_n = _normalize_table(table)
    edges = jnp.full((_EDGE_PAD,), jnp.inf, dtype=jnp.float32)
    edges = edges.at[: _VOCAB + 1].set(bin_edges)
    emb_t, toks = _sc_lookup(ts_values, table_n.reshape(-1), edges)
    return (emb_t.T, toks)
